# baseline (speedup 1.0000x reference)
import functools

import jax
import jax.numpy as jnp
from jax import lax
from jax.experimental import pallas as pl
from jax.experimental.pallas import tpu as pltpu
from jax.experimental.pallas import tpu_sc as plsc

N_PTS = 200000
FDIM = 64
TDIM = 64
MT = 256
KNN = 16
HID = 64
BN = 2000
NB = 100
BN3 = 2048
NB3 = 100
NP = BN3 * NB3
R2 = 500
C2 = 400
FBIG = 3.0e38
NEG = -3.0e38


def _k1_body(feat_ref, c4_ref, gn_ref, w1_ref, b1_ref, w2_ref, b2_ref,
             wi1f_ref, wi1c_ref, bi1_ref, lng_ref, lnb_ref, wi2_ref,
             bi2_ref, wi3t_ref, bi3_ref, tmp_ref,
             pert_ref, bn_ref):
    x = feat_ref[:, :]
    c4 = c4_ref[:, :]
    h = jnp.maximum(jnp.dot(x, w1_ref[:, :], preferred_element_type=jnp.float32)
                    + b1_ref[:, :], 0.0)
    feats = jnp.dot(h, w2_ref[:, :], preferred_element_type=jnp.float32) + b2_ref[:, :]
    hi = jnp.dot(feats, wi1f_ref[:, :], preferred_element_type=jnp.float32) \
        + jnp.dot(c4, wi1c_ref[:, :], preferred_element_type=jnp.float32) \
        + bi1_ref[:, :]
    hi = jnp.maximum(hi, 0.0)
    mu = jnp.mean(hi, axis=1, keepdims=True)
    d = hi - mu
    var = jnp.mean(d * d, axis=1, keepdims=True)
    hi = d * jax.lax.rsqrt(var + 1e-5) * lng_ref[:, :] + lnb_ref[:, :]
    hi = jnp.maximum(jnp.dot(hi, wi2_ref[:, :], preferred_element_type=jnp.float32)
                     + bi2_ref[:, :], 0.0)
    imp = jnp.sum(hi * wi3t_ref[:, :], axis=1, keepdims=True) + bi3_ref[:, :]
    pert_ref[:, :] = (imp + gn_ref[:, :]) / tmp_ref[:, :]
    bn_ref[:, :] = jnp.sum(c4 * c4, axis=1, keepdims=True)


def _run_k1(features, coords4, gnoise, W1, b1, W2, b2, Wi1f, Wi1c,
            bi1, ln_g, ln_b, Wi2, bi2, Wi3t, bi3, tempc):
    full = lambda shp: pl.BlockSpec(shp, lambda i: (0, 0))
    return pl.pallas_call(
        _k1_body,
        grid=(NB,),
        in_specs=[
            pl.BlockSpec((BN, FDIM), lambda i: (i, 0)),
            pl.BlockSpec((BN, 4), lambda i: (i, 0)),
            pl.BlockSpec((BN, 1), lambda i: (i, 0)),
            full((FDIM, 64)), full((1, 64)), full((64, TDIM)), full((1, TDIM)),
            full((TDIM, HID)), full((4, HID)), full((1, HID)),
            full((1, HID)), full((1, HID)), full((HID, HID)), full((1, HID)),
            full((1, HID)), full((1, 1)), full((1, 1)),
        ],
        out_specs=[
            pl.BlockSpec((BN, 1), lambda i: (i, 0)),
            pl.BlockSpec((BN, 1), lambda i: (i, 0)),
        ],
        out_shape=[
            jax.ShapeDtypeStruct((N_PTS, 1), jnp.float32),
            jax.ShapeDtypeStruct((N_PTS, 1), jnp.float32),
        ],
    )(features, coords4, gnoise, W1, b1, W2, b2, Wi1f, Wi1c, bi1,
      ln_g, ln_b, Wi2, bi2, Wi3t, bi3, tempc)


def _k2_body(pert_ref, c4_ref, sel_ref, cen_ref, a_ref, rm_ref):
    a_ref[:, :] = pert_ref[:, :]
    rm_ref[:, :] = jnp.max(pert_ref[:, :], axis=1, keepdims=True)

    def step(i, _):
        rmv = rm_ref[:, :]
        m = jnp.max(rmv)
        riota = lax.broadcasted_iota(jnp.int32, (R2, 1), 0)
        r = jnp.min(jnp.where(rmv >= m, riota, R2))
        row = a_ref[pl.ds(r, 1), :]
        ciota = lax.broadcasted_iota(jnp.int32, (1, C2), 1)
        c = jnp.min(jnp.where(row >= m, ciota, C2))
        flat = r * C2 + c
        sel_ref[pl.ds(i, 1), :] = jnp.full((1, 1), flat, jnp.int32)
        hot = ciota == c
        cen_ref[pl.ds(i, 1), :] = jnp.concatenate(
            [jnp.sum(jnp.where(hot, c4_ref[k, pl.ds(r, 1), :], 0.0),
                     axis=1, keepdims=True) for k in range(4)], axis=1)
        nrow = jnp.where(hot, NEG, row)
        a_ref[pl.ds(r, 1), :] = nrow
        rm_ref[pl.ds(r, 1), :] = jnp.max(nrow, axis=1, keepdims=True)
        return 0

    lax.fori_loop(0, MT, step, 0)


def _run_k2(pert_rc, c4_rc):
    return pl.pallas_call(
        _k2_body,
        in_specs=[
            pl.BlockSpec((R2, C2), lambda: (0, 0)),
            pl.BlockSpec((4, R2, C2), lambda: (0, 0, 0)),
        ],
        out_specs=[
            pl.BlockSpec((MT, 1), lambda: (0, 0)),
            pl.BlockSpec((MT, 4), lambda: (0, 0)),
        ],
        out_shape=[
            jax.ShapeDtypeStruct((MT, 1), jnp.int32),
            jax.ShapeDtypeStruct((MT, 4), jnp.float32),
        ],
        scratch_shapes=[
            pltpu.VMEM((R2, C2), jnp.float32),
            pltpu.VMEM((R2, 1), jnp.float32),
        ],
    )(pert_rc, c4_rc)


def _k3_body(cen_ref, c4t_ref, bn_ref, knn_ref, bd_ref, bi_ref, d2_ref,
             flag_ref):
    i = pl.program_id(0)

    @pl.when(i == 0)
    def _init():
        bd_ref[:, :] = jnp.full((MT, KNN), FBIG, jnp.float32)
        bi_ref[:, :] = jnp.zeros((MT, KNN), jnp.int32)

    cen = cen_ref[:, :]
    cn = jnp.sum(cen * cen, axis=1, keepdims=True)
    prod = jax.lax.dot_general(cen, c4t_ref[:, :],
                               (((1,), (0,)), ((), ())),
                               preferred_element_type=jnp.float32)
    d2 = jnp.maximum(cn + bn_ref[:, :] - 2.0 * prod, 0.0)
    d2_ref[:, :] = d2
    base = i * BN3
    flag_ref[0] = 1
    ciota = lax.broadcasted_iota(jnp.int32, (MT, BN3), 1)
    kiota = lax.broadcasted_iota(jnp.int32, (MT, KNN), 1)

    def step(_, __):
        @pl.when(flag_ref[0] > 0)
        def _active():
            bd = bd_ref[:, :]
            t = jnp.max(bd, axis=1, keepdims=True)
            dv = d2_ref[:, :]
            m = jnp.min(dv, axis=1, keepdims=True)
            need = m < t
            c = jnp.min(jnp.where(dv <= m, ciota, BN3), axis=1, keepdims=True)
            d2_ref[:, :] = jnp.where((ciota == c) & need, FBIG, dv)
            pos = jnp.min(jnp.where(bd >= t, kiota, KNN), axis=1, keepdims=True)
            repl = (kiota == pos) & need
            bd_ref[:, :] = jnp.where(repl, m, bd)
            bi_ref[:, :] = jnp.where(repl, base + c, bi_ref[:, :])
            flag_ref[0] = jnp.sum(need.astype(jnp.int32))

        return 0

    lax.fori_loop(0, KNN, step, 0)

    @pl.when(i == NB3 - 1)
    def _fin():
        knn_ref[:, :] = bi_ref[:, :]


def _run_k3(centroids, coords4T, bn_row):
    return pl.pallas_call(
        _k3_body,
        grid=(NB3,),
        in_specs=[
            pl.BlockSpec((MT, 4), lambda i: (0, 0)),
            pl.BlockSpec((4, BN3), lambda i: (0, i)),
            pl.BlockSpec((1, BN3), lambda i: (0, i)),
        ],
        out_specs=pl.BlockSpec((MT, KNN), lambda i: (0, 0)),
        out_shape=jax.ShapeDtypeStruct((MT, KNN), jnp.int32),
        scratch_shapes=[
            pltpu.VMEM((MT, KNN), jnp.float32),
            pltpu.VMEM((MT, KNN), jnp.int32),
            pltpu.VMEM((MT, BN3), jnp.float32),
            pltpu.SMEM((1,), jnp.int32),
        ],
        compiler_params=pltpu.CompilerParams(
            dimension_semantics=("arbitrary",)),
    )(centroids, coords4T, bn_row)


def _sc_gather(table, idx):
    B = idx.shape[0]
    D = table.shape[1]
    info = plsc.get_sparse_core_info()
    nw = info.num_cores * info.num_subcores
    bpw = B // nw
    mesh = plsc.VectorSubcoreMesh(core_axis_name="c", subcore_axis_name="s")

    @functools.partial(
        pl.kernel, mesh=mesh,
        out_type=jax.ShapeDtypeStruct((B, D), jnp.float32),
        scratch_types=[
            pltpu.VMEM((bpw,), jnp.int32),
            pltpu.VMEM((bpw, D), jnp.float32),
            pltpu.SemaphoreType.DMA,
        ],
    )
    def k(table_hbm, idx_hbm, out_hbm, idx_v, rows_v, sem):
        wid = lax.axis_index("s") * info.num_cores + lax.axis_index("c")
        base = wid * bpw
        pltpu.sync_copy(idx_hbm.at[pl.ds(base, bpw)], idx_v)
        pltpu.async_copy(table_hbm.at[idx_v], rows_v, sem).wait()
        pltpu.sync_copy(rows_v, out_hbm.at[pl.ds(base, bpw)])

    return k(table, idx)


def _k5_body(g_ref, par_ref, cen_ref, cent_ref, w1l_ref, w1r_ref, b1_ref,
             w2_ref, b2_ref, wn1_ref, bn1_ref, wn2_ref, bn2_ref,
             tok_ref, cout_ref):
    g = g_ref[:, :]
    par = par_ref[:, :]
    hl = jnp.dot(g, w1l_ref[:, :], preferred_element_type=jnp.float32)
    hr = jnp.dot(g, w1r_ref[:, :], preferred_element_type=jnp.float32)
    h = jnp.maximum(jnp.where(par > 0, hr, hl) + b1_ref[:, :], 0.0)
    fe = jnp.dot(h, w2_ref[:, :], preferred_element_type=jnp.float32) \
        + b2_ref[:, :]
    pooled = fe[0:MT, :]
    for j in range(1, KNN):
        pooled = jnp.maximum(pooled, fe[j * MT:(j + 1) * MT, :])
    h2 = jnp.maximum(jnp.dot(pooled, wn1_ref[:, :],
                             preferred_element_type=jnp.float32)
                     + bn1_ref[:, :], 0.0)
    tok = jnp.dot(h2, wn2_ref[:, :], preferred_element_type=jnp.float32) \
        + bn2_ref[:, :]
    tj = cen_ref[:, 3:4]
    ti = cent_ref[3:4, :]
    jio = lax.broadcasted_iota(jnp.int32, (MT, 1), 0)
    iio = lax.broadcasted_iota(jnp.int32, (1, MT), 1)
    cmp = (tj < ti) | ((tj == ti) & (jio < iio))
    rank = jnp.sum(cmp.astype(jnp.int32), axis=0, keepdims=True)
    rio = lax.broadcasted_iota(jnp.int32, (MT, MT), 0)
    perm = (rio == rank).astype(jnp.float32)
    tok_ref[:, :] = jnp.dot(perm, tok, preferred_element_type=jnp.float32)
    cout_ref[:, :] = jnp.dot(perm, cen_ref[:, :],
                             preferred_element_type=jnp.float32)


def _run_k5(gathered, parity, centroids, centroidsT,
            W1L, W1R, b1, W2, b2, Wn1, bn1, Wn2, bn2):
    full2 = lambda shp: pl.BlockSpec(shp, lambda: tuple(0 for _ in shp))
    return pl.pallas_call(
        _k5_body,
        in_specs=[
            full2((MT * KNN, 2 * FDIM)), full2((MT * KNN, 1)),
            full2((MT, 4)), full2((4, MT)),
            full2((2 * FDIM, 64)), full2((2 * FDIM, 64)), full2((1, 64)),
            full2((64, TDIM)), full2((1, TDIM)),
            full2((TDIM, TDIM)), full2((1, TDIM)),
            full2((TDIM, TDIM)), full2((1, TDIM)),
        ],
        out_specs=[full2((MT, TDIM)), full2((MT, 4))],
        out_shape=[
            jax.ShapeDtypeStruct((MT, TDIM), jnp.float32),
            jax.ShapeDtypeStruct((MT, 4), jnp.float32),
        ],
    )(gathered, parity, centroids, centroidsT,
      W1L, W1R, b1, W2, b2, Wn1, bn1, Wn2, bn2)


def kernel(coordinates, features, W1, b1, W2, b2, Wi1, bi1, ln_g, ln_b,
           Wi2, bi2, Wi3, bi3, Wn1, bn1, Wn2, bn2, temp):
    n = coordinates.shape[0]
    coords4 = coordinates[:, 1:5]
    u = jax.random.uniform(jax.random.key(42), (n,), dtype=jnp.float32)
    gnoise = -jnp.log(-jnp.log(u + 1e-10) + 1e-10)
    temperature = jax.nn.softplus(temp) + 0.1

    pert, bn_col = _run_k1(
        features, coords4, gnoise[:, None],
        W1, b1[None, :], W2, b2[None, :],
        Wi1[:TDIM], Wi1[TDIM:], bi1[None, :], ln_g[None, :], ln_b[None, :],
        Wi2, bi2[None, :], Wi3.T, bi3[None, :],
        jnp.reshape(temperature, (1, 1)))

    coords4T = coords4.T
    sel, centroids = _run_k2(jnp.reshape(pert, (R2, C2)),
                             jnp.reshape(coords4T, (4, R2, C2)))

    c4t_pad = jnp.pad(coords4T, ((0, 0), (0, NP - n)), constant_values=1e3)
    bn_pad = jnp.pad(jnp.reshape(bn_col, (1, N_PTS)), ((0, 0), (0, NP - n)),
                     constant_values=1e30)
    knn = _run_k3(centroids, c4t_pad, bn_pad)

    idx_jmajor = jnp.reshape(knn.T, (MT * KNN,))
    half_table = jnp.reshape(features, (n // 2, 2 * FDIM))
    gathered = _sc_gather(half_table, idx_jmajor // 2)
    parity = (idx_jmajor % 2).astype(jnp.int32)[:, None]

    zpad = jnp.zeros_like(W1)
    W1L = jnp.concatenate([W1, zpad], axis=0)
    W1R = jnp.concatenate([zpad, W1], axis=0)
    tokens_s, cen_s = _run_k5(gathered, parity, centroids, centroids.T,
                              W1L, W1R, b1[None, :], W2, b2[None, :],
                              Wn1, bn1[None, :], Wn2, bn2[None, :])

    masks = jnp.ones((1, MT), dtype=bool)
    return tokens_s[None], cen_s[None], masks


# speedup vs baseline: 1.3677x; 1.3677x over previous
import functools

import jax
import jax.numpy as jnp
from jax import lax
from jax.experimental import pallas as pl
from jax.experimental.pallas import tpu as pltpu
from jax.experimental.pallas import tpu_sc as plsc

N_PTS = 200000
FDIM = 64
TDIM = 64
MT = 256
KNN = 16
PK = 4
NQ = N_PTS // PK
BQ = 1000
NB = 50
BN3 = 2048
NB3 = 100
NP = BN3 * NB3
R2 = 100
C2 = 2000
FBIG = 3.0e38
NEG = -3.0e38


def _k1_body(f4_ref, c16_ref, gn_ref, w1_ref, b1_ref, w2_ref, b2_ref,
             wi1f_ref, wi1c_ref, bi1_ref, lng_ref, lnb_ref, wi2_ref,
             bi2_ref, wi3_ref, bi3_ref, mavg_ref, ssum_ref, tmp_ref,
             pert_ref, bn_ref):
    x = f4_ref[:, :]
    c16 = c16_ref[:, :]
    h = jnp.maximum(jnp.dot(x, w1_ref[:, :], preferred_element_type=jnp.float32)
                    + b1_ref[:, :], 0.0)
    f = jnp.dot(h, w2_ref[:, :], preferred_element_type=jnp.float32) + b2_ref[:, :]
    hi = jnp.dot(f, wi1f_ref[:, :], preferred_element_type=jnp.float32) \
        + jnp.dot(c16, wi1c_ref[:, :], preferred_element_type=jnp.float32) \
        + bi1_ref[:, :]
    hi = jnp.maximum(hi, 0.0)
    mu = jnp.dot(hi, mavg_ref[:, :], preferred_element_type=jnp.float32)
    d = hi - mu
    var = jnp.dot(d * d, mavg_ref[:, :], preferred_element_type=jnp.float32)
    hi = d * jax.lax.rsqrt(var + 1e-5) * lng_ref[:, :] + lnb_ref[:, :]
    hi = jnp.maximum(jnp.dot(hi, wi2_ref[:, :], preferred_element_type=jnp.float32)
                     + bi2_ref[:, :], 0.0)
    imp = jnp.dot(hi, wi3_ref[:, :], preferred_element_type=jnp.float32) \
        + bi3_ref[:, :]
    pert_ref[:, :] = (imp + gn_ref[:, :]) / tmp_ref[:, :]
    bn_ref[:, :] = jnp.dot(c16 * c16, ssum_ref[:, :],
                           preferred_element_type=jnp.float32)


def _run_k1(f4, c16, gn4, W1b, b1t, W2b, b2t, Wi1fb, Wi1cb, bi1t,
            lngt, lnbt, Wi2b, bi2t, wi3b, bi3t, mavg, ssum, tempc):
    full = lambda shp: pl.BlockSpec(shp, lambda i: (0, 0))
    return pl.pallas_call(
        _k1_body,
        grid=(NB,),
        in_specs=[
            pl.BlockSpec((BQ, PK * FDIM), lambda i: (i, 0)),
            pl.BlockSpec((BQ, PK * 4), lambda i: (i, 0)),
            pl.BlockSpec((BQ, PK), lambda i: (i, 0)),
            full((PK * FDIM, PK * 64)), full((1, PK * 64)),
            full((PK * 64, PK * TDIM)), full((1, PK * TDIM)),
            full((PK * TDIM, PK * 64)), full((PK * 4, PK * 64)),
            full((1, PK * 64)), full((1, PK * 64)), full((1, PK * 64)),
            full((PK * 64, PK * 64)), full((1, PK * 64)),
            full((PK * 64, PK)), full((1, PK)),
            full((PK * 64, PK * 64)), full((PK * 4, PK)),
            full((1, 1)),
        ],
        out_specs=[
            pl.BlockSpec((BQ, PK), lambda i: (i, 0)),
            pl.BlockSpec((BQ, PK), lambda i: (i, 0)),
        ],
        out_shape=[
            jax.ShapeDtypeStruct((NQ, PK), jnp.float32),
            jax.ShapeDtypeStruct((NQ, PK), jnp.float32),
        ],
    )(f4, c16, gn4, W1b, b1t, W2b, b2t, Wi1fb, Wi1cb, bi1t,
      lngt, lnbt, Wi2b, bi2t, wi3b, bi3t, mavg, ssum, tempc)


def _k2_body(pert_ref, c4_ref, sel_ref, cen_ref, a_ref, rm_ref):
    a_ref[:, :] = pert_ref[:, :]
    rm_ref[:, :] = jnp.max(pert_ref[:, :], axis=1, keepdims=True)
    riota = lax.broadcasted_iota(jnp.int32, (R2, 1), 0)
    ciota = lax.broadcasted_iota(jnp.int32, (1, C2), 1)

    def step(i, _):
        rmv = rm_ref[:, :]
        m = jnp.max(rmv)
        r = jnp.min(jnp.where(rmv >= m, riota, R2))
        row = a_ref[pl.ds(r, 1), :]
        c = jnp.min(jnp.where(row >= m, ciota, C2))
        flat = r * C2 + c
        sel_ref[pl.ds(i, 1), :] = jnp.full((1, 1), flat, jnp.int32)
        nrow = jnp.where(ciota == c, NEG, row)
        a_ref[pl.ds(r, 1), :] = nrow
        rm_ref[pl.ds(r, 1), :] = jnp.max(nrow, axis=1, keepdims=True)
        return 0

    lax.fori_loop(0, MT, step, 0)

    selv = sel_ref[:, :]
    ri = selv // C2
    ci = selv - ri * C2
    rio2 = lax.broadcasted_iota(jnp.int32, (MT, R2), 1)
    cio2 = lax.broadcasted_iota(jnp.int32, (MT, C2), 1)
    rowhot = (rio2 == ri).astype(jnp.float32)
    colhot = (cio2 == ci).astype(jnp.float32)
    cen = []
    for k in range(4):
        tmpk = jnp.dot(rowhot, c4_ref[k, :, :],
                       preferred_element_type=jnp.float32)
        cen.append(jnp.sum(tmpk * colhot, axis=1, keepdims=True))
    cen_ref[:, :] = jnp.concatenate(cen, axis=1)


def _run_k2(pert_rc, c4_rc):
    return pl.pallas_call(
        _k2_body,
        in_specs=[
            pl.BlockSpec((R2, C2), lambda: (0, 0)),
            pl.BlockSpec((4, R2, C2), lambda: (0, 0, 0)),
        ],
        out_specs=[
            pl.BlockSpec((MT, 1), lambda: (0, 0)),
            pl.BlockSpec((MT, 4), lambda: (0, 0)),
        ],
        out_shape=[
            jax.ShapeDtypeStruct((MT, 1), jnp.int32),
            jax.ShapeDtypeStruct((MT, 4), jnp.float32),
        ],
        scratch_shapes=[
            pltpu.VMEM((R2, C2), jnp.float32),
            pltpu.VMEM((R2, 1), jnp.float32),
        ],
    )(pert_rc, c4_rc)


def _k3_body(cen_ref, c4t_ref, bn_ref, knn_ref, bd_ref, bi_ref, d2_ref,
             flag_ref):
    i = pl.program_id(0)

    @pl.when(i == 0)
    def _init():
        bd_ref[:, :] = jnp.full((MT, KNN), FBIG, jnp.float32)
        bi_ref[:, :] = jnp.zeros((MT, KNN), jnp.int32)

    cen = cen_ref[:, :]
    cn = jnp.sum(cen * cen, axis=1, keepdims=True)
    prod = jax.lax.dot_general(cen, c4t_ref[:, :],
                               (((1,), (0,)), ((), ())),
                               preferred_element_type=jnp.float32)
    d2 = jnp.maximum(cn + bn_ref[:, :] - 2.0 * prod, 0.0)
    d2_ref[:, :] = d2
    base = i * BN3
    flag_ref[0] = 1
    ciota = lax.broadcasted_iota(jnp.int32, (MT, BN3), 1)
    kiota = lax.broadcasted_iota(jnp.int32, (MT, KNN), 1)

    def step(_, __):
        @pl.when(flag_ref[0] > 0)
        def _active():
            bd = bd_ref[:, :]
            t = jnp.max(bd, axis=1, keepdims=True)
            dv = d2_ref[:, :]
            m = jnp.min(dv, axis=1, keepdims=True)
            need = m < t
            c = jnp.min(jnp.where(dv <= m, ciota, BN3), axis=1, keepdims=True)
            d2_ref[:, :] = jnp.where((ciota == c) & need, FBIG, dv)
            pos = jnp.min(jnp.where(bd >= t, kiota, KNN), axis=1, keepdims=True)
            repl = (kiota == pos) & need
            bd_ref[:, :] = jnp.where(repl, m, bd)
            bi_ref[:, :] = jnp.where(repl, base + c, bi_ref[:, :])
            flag_ref[0] = jnp.sum(need.astype(jnp.int32))

        return 0

    lax.fori_loop(0, KNN, step, 0)

    @pl.when(i == NB3 - 1)
    def _fin():
        knn_ref[:, :] = bi_ref[:, :]


def _run_k3(centroids, coords4T, bn_row):
    return pl.pallas_call(
        _k3_body,
        grid=(NB3,),
        in_specs=[
            pl.BlockSpec((MT, 4), lambda i: (0, 0)),
            pl.BlockSpec((4, BN3), lambda i: (0, i)),
            pl.BlockSpec((1, BN3), lambda i: (0, i)),
        ],
        out_specs=pl.BlockSpec((MT, KNN), lambda i: (0, 0)),
        out_shape=jax.ShapeDtypeStruct((MT, KNN), jnp.int32),
        scratch_shapes=[
            pltpu.VMEM((MT, KNN), jnp.float32),
            pltpu.VMEM((MT, KNN), jnp.int32),
            pltpu.VMEM((MT, BN3), jnp.float32),
            pltpu.SMEM((1,), jnp.int32),
        ],
        compiler_params=pltpu.CompilerParams(
            dimension_semantics=("arbitrary",)),
    )(centroids, coords4T, bn_row)


def _sc_gather(table, idx):
    B = idx.shape[0]
    D = table.shape[1]
    info = plsc.get_sparse_core_info()
    nw = info.num_cores * info.num_subcores
    bpw = B // nw
    mesh = plsc.VectorSubcoreMesh(core_axis_name="c", subcore_axis_name="s")

    @functools.partial(
        pl.kernel, mesh=mesh,
        out_type=jax.ShapeDtypeStruct((B, D), jnp.float32),
        scratch_types=[
            pltpu.VMEM((bpw,), jnp.int32),
            pltpu.VMEM((bpw, D), jnp.float32),
            pltpu.SemaphoreType.DMA,
        ],
    )
    def k(table_hbm, idx_hbm, out_hbm, idx_v, rows_v, sem):
        wid = lax.axis_index("s") * info.num_cores + lax.axis_index("c")
        base = wid * bpw
        pltpu.sync_copy(idx_hbm.at[pl.ds(base, bpw)], idx_v)
        pltpu.async_copy(table_hbm.at[idx_v], rows_v, sem).wait()
        pltpu.sync_copy(rows_v, out_hbm.at[pl.ds(base, bpw)])

    return k(table, idx)


def _k5_body(g_ref, par_ref, cen_ref, cent_ref, w1q0_ref, w1q1_ref,
             w1q2_ref, w1q3_ref, b1_ref, w2_ref, b2_ref,
             wn1_ref, bn1_ref, wn2_ref, bn2_ref,
             tok_ref, cout_ref):
    g = g_ref[:, :]
    par = par_ref[:, :]
    h0 = jnp.dot(g, w1q0_ref[:, :], preferred_element_type=jnp.float32)
    h1 = jnp.dot(g, w1q1_ref[:, :], preferred_element_type=jnp.float32)
    h2 = jnp.dot(g, w1q2_ref[:, :], preferred_element_type=jnp.float32)
    h3 = jnp.dot(g, w1q3_ref[:, :], preferred_element_type=jnp.float32)
    hsel = jnp.where(par <= 1, jnp.where(par == 0, h0, h1),
                     jnp.where(par == 2, h2, h3))
    h = jnp.maximum(hsel + b1_ref[:, :], 0.0)
    fe = jnp.dot(h, w2_ref[:, :], preferred_element_type=jnp.float32) \
        + b2_ref[:, :]
    pooled = fe[0:MT, :]
    for j in range(1, KNN):
        pooled = jnp.maximum(pooled, fe[j * MT:(j + 1) * MT, :])
    hn = jnp.maximum(jnp.dot(pooled, wn1_ref[:, :],
                             preferred_element_type=jnp.float32)
                     + bn1_ref[:, :], 0.0)
    tok = jnp.dot(hn, wn2_ref[:, :], preferred_element_type=jnp.float32) \
        + bn2_ref[:, :]
    tj = cen_ref[:, 3:4]
    ti = cent_ref[3:4, :]
    jio = lax.broadcasted_iota(jnp.int32, (MT, 1), 0)
    iio = lax.broadcasted_iota(jnp.int32, (1, MT), 1)
    cmp = (tj < ti) | ((tj == ti) & (jio < iio))
    rank = jnp.sum(cmp.astype(jnp.int32), axis=0, keepdims=True)
    rio = lax.broadcasted_iota(jnp.int32, (MT, MT), 0)
    perm = (rio == rank).astype(jnp.float32)
    tok_ref[:, :] = jnp.dot(perm, tok, preferred_element_type=jnp.float32)
    cout_ref[:, :] = jnp.dot(perm, cen_ref[:, :],
                             preferred_element_type=jnp.float32)


def _run_k5(gathered, parity, centroids, centroidsT,
            W1Q, b1, W2, b2, Wn1, bn1, Wn2, bn2):
    full2 = lambda shp: pl.BlockSpec(shp, lambda: tuple(0 for _ in shp))
    return pl.pallas_call(
        _k5_body,
        in_specs=[
            full2((MT * KNN, PK * FDIM)), full2((MT * KNN, 1)),
            full2((MT, 4)), full2((4, MT)),
            full2((PK * FDIM, 64)), full2((PK * FDIM, 64)),
            full2((PK * FDIM, 64)), full2((PK * FDIM, 64)),
            full2((1, 64)),
            full2((64, TDIM)), full2((1, TDIM)),
            full2((TDIM, TDIM)), full2((1, TDIM)),
            full2((TDIM, TDIM)), full2((1, TDIM)),
        ],
        out_specs=[full2((MT, TDIM)), full2((MT, 4))],
        out_shape=[
            jax.ShapeDtypeStruct((MT, TDIM), jnp.float32),
            jax.ShapeDtypeStruct((MT, 4), jnp.float32),
        ],
    )(gathered, parity, centroids, centroidsT,
      W1Q[0], W1Q[1], W1Q[2], W1Q[3], b1, W2, b2, Wn1, bn1, Wn2, bn2)


def kernel(coordinates, features, W1, b1, W2, b2, Wi1, bi1, ln_g, ln_b,
           Wi2, bi2, Wi3, bi3, Wn1, bn1, Wn2, bn2, temp):
    n = coordinates.shape[0]
    coords4 = coordinates[:, 1:5]
    u = jax.random.uniform(jax.random.key(42), (n,), dtype=jnp.float32)
    gnoise = -jnp.log(-jnp.log(u + 1e-10) + 1e-10)
    temperature = jax.nn.softplus(temp) + 0.1

    eye = jnp.eye(PK, dtype=jnp.float32)
    kron = jnp.kron
    W1b = kron(eye, W1)
    W2b = kron(eye, W2)
    Wi1fb = kron(eye, Wi1[:TDIM])
    Wi1cb = kron(eye, Wi1[TDIM:])
    Wi2b = kron(eye, Wi2)
    wi3b = kron(eye, Wi3)
    mavg = kron(eye, jnp.full((64, 64), 1.0 / 64, jnp.float32))
    ssum = kron(eye, jnp.ones((4, 1), jnp.float32))
    tile4 = lambda v: jnp.tile(v, (PK,))[None, :]
    b1t, b2t, bi1t = tile4(b1), tile4(b2), tile4(bi1)
    lngt, lnbt, bi2t = tile4(ln_g), tile4(ln_b), tile4(bi2)
    bi3t = jnp.tile(bi3, (PK,))[None, :]

    f4 = jnp.reshape(features, (NQ, PK * FDIM))
    c16 = jnp.reshape(coords4, (NQ, PK * 4))
    gn4 = jnp.reshape(gnoise, (NQ, PK))

    pert4, bn4 = _run_k1(f4, c16, gn4, W1b, b1t, W2b, b2t, Wi1fb, Wi1cb,
                         bi1t, lngt, lnbt, Wi2b, bi2t, wi3b, bi3t,
                         mavg, ssum, jnp.reshape(temperature, (1, 1)))

    coords4T = coords4.T
    sel, centroids = _run_k2(jnp.reshape(pert4, (R2, C2)),
                             jnp.reshape(coords4T, (4, R2, C2)))

    c4t_pad = jnp.pad(coords4T, ((0, 0), (0, NP - n)), constant_values=1e3)
    bn_pad = jnp.pad(jnp.reshape(bn4, (1, N_PTS)), ((0, 0), (0, NP - n)),
                     constant_values=1e30)
    knn = _run_k3(centroids, c4t_pad, bn_pad)

    idx_jmajor = jnp.reshape(knn.T, (MT * KNN,))
    gathered = _sc_gather(f4, idx_jmajor // PK)
    parity = (idx_jmajor % PK).astype(jnp.int32)[:, None]

    W1Q = [kron(eye[:, q:q + 1], W1) for q in range(PK)]
    tokens_s, cen_s = _run_k5(gathered, parity, centroids, centroids.T,
                              W1Q, b1[None, :], W2, b2[None, :],
                              Wn1, bn1[None, :], Wn2, bn2[None, :])

    masks = jnp.ones((1, MT), dtype=bool)
    return tokens_s[None], cen_s[None], masks
